# baseline (speedup 1.0000x reference)
import jax
import jax.numpy as jnp
from jax import lax
from jax.experimental import pallas as pl
from jax.experimental.pallas import tpu as pltpu
from jax.experimental.pallas import tpu_sc as plsc

_NUM_FIELDS = 26
_VOCAB = 100000
_EMB = 32
_BATCH = 16384
_DENSE = 13
_OUT_W = _NUM_FIELDS * _EMB + _DENSE

_NC = 2
_NS = 16
_NW = _NC * _NS
_B_PER_W = _BATCH // _NW
_CH = 16
_N_CHUNKS = _B_PER_W // _CH


def _body(*refs):
    cats = refs[:_NUM_FIELDS]
    tables = refs[_NUM_FIELDS:2 * _NUM_FIELDS]
    dense = refs[2 * _NUM_FIELDS]
    out = refs[2 * _NUM_FIELDS + 1]
    idx_all = refs[2 * _NUM_FIELDS + 2]
    stage = refs[2 * _NUM_FIELDS + 3]
    dbuf = refs[2 * _NUM_FIELDS + 4]
    dummy = refs[2 * _NUM_FIELDS + 5]
    sem = refs[2 * _NUM_FIELDS + 6]
    wsem = refs[2 * _NUM_FIELDS + 7]

    wid = lax.axis_index("s") * _NC + lax.axis_index("c")

    for f in range(_NUM_FIELDS):
        pltpu.sync_copy(cats[f].at[wid], idx_all.at[f])

    lanes = lax.iota(jnp.int32, 16)
    dmask = lanes < _DENSE

    def fire(c, bank):
        for f in range(_NUM_FIELDS):
            vec = idx_all[f, pl.ds(c * _CH, 16)]
            for lane in range(16):
                t = vec[lane]
                pltpu.async_copy(
                    tables[f].at[pl.ds(t, 1)],
                    stage.at[bank, pl.ds(lane, 1), pl.ds(f * _EMB, _EMB)],
                    sem.at[bank])

    def complete(c, bank):
        pltpu.sync_copy(dense.at[pl.ds(wid * _B_PER_W + c * _CH, _CH)], dbuf)

        @pl.loop(0, _CH)
        def _dk(r):
            r_vec = jnp.full((16,), r, jnp.int32)
            c_vec = jnp.where(dmask, lanes, 0)
            vals = plsc.load_gather(dbuf, [r_vec, c_vec], mask=dmask)
            plsc.store_scatter(stage.at[bank],
                               [r_vec, c_vec + _NUM_FIELDS * _EMB],
                               vals, mask=dmask)

        pltpu.make_async_copy(
            tables[0].at[pl.ds(0, _CH * _NUM_FIELDS)], dummy,
            sem.at[bank]).wait()
        pltpu.async_copy(stage.at[bank],
                         out.at[pl.ds(wid * _B_PER_W + c * _CH, _CH)],
                         wsem.at[bank])

    @pl.loop(0, _N_CHUNKS)
    def _chunk(c):
        bank = jnp.bitwise_and(c, 1)

        @pl.when(c >= 2)
        def _():
            pltpu.make_async_copy(
                out.at[pl.ds(0, _CH)], stage.at[bank], wsem.at[bank]).wait()

        fire(c, bank)

        @pl.when(c >= 1)
        def _():
            complete(c - 1, 1 - bank)

    complete(_N_CHUNKS - 1, jnp.bitwise_and(_N_CHUNKS - 1, 1))
    for b in range(2):
        pltpu.make_async_copy(
            out.at[pl.ds(0, _CH)], stage.at[b], wsem.at[b]).wait()


def kernel(cat_0, cat_1, cat_2, cat_3, cat_4, cat_5, cat_6, cat_7, cat_8, cat_9, cat_10, cat_11, cat_12, cat_13, cat_14, cat_15, cat_16, cat_17, cat_18, cat_19, cat_20, cat_21, cat_22, cat_23, cat_24, cat_25, table_0, table_1, table_2, table_3, table_4, table_5, table_6, table_7, table_8, table_9, table_10, table_11, table_12, table_13, table_14, table_15, table_16, table_17, table_18, table_19, table_20, table_21, table_22, table_23, table_24, table_25, dense):
    cats = [cat_0, cat_1, cat_2, cat_3, cat_4, cat_5, cat_6, cat_7, cat_8,
            cat_9, cat_10, cat_11, cat_12, cat_13, cat_14, cat_15, cat_16,
            cat_17, cat_18, cat_19, cat_20, cat_21, cat_22, cat_23, cat_24,
            cat_25]
    tables = [table_0, table_1, table_2, table_3, table_4, table_5, table_6,
              table_7, table_8, table_9, table_10, table_11, table_12,
              table_13, table_14, table_15, table_16, table_17, table_18,
              table_19, table_20, table_21, table_22, table_23, table_24,
              table_25]
    cats_r = [c.reshape(_NW, _B_PER_W) for c in cats]
    dense_r = dense.reshape(_BATCH, _DENSE)

    mesh = plsc.VectorSubcoreMesh(core_axis_name="c", subcore_axis_name="s")
    out = pl.kernel(
        _body,
        out_type=jax.ShapeDtypeStruct((_BATCH, _OUT_W), jnp.float32),
        mesh=mesh,
        scratch_types=[
            pltpu.VMEM((_NUM_FIELDS, _B_PER_W), jnp.int32),
            pltpu.VMEM((2, _CH, _OUT_W), jnp.float32),
            pltpu.VMEM((_CH, _DENSE), jnp.float32),
            pltpu.VMEM((_CH * _NUM_FIELDS, _EMB), jnp.float32),
            pltpu.SemaphoreType.DMA((2,)),
            pltpu.SemaphoreType.DMA((2,)),
        ],
        compiler_params=pltpu.CompilerParams(
            disable_bounds_checks=True, needs_layout_passes=False,
            use_tc_tiling_on_sc=False),
    )(*cats_r, *tables, dense_r)
    return out.reshape(_BATCH, 1, _OUT_W)


# speedup vs baseline: 1.0209x; 1.0209x over previous
import jax
import jax.numpy as jnp
from jax import lax
from jax.experimental import pallas as pl
from jax.experimental.pallas import tpu as pltpu
from jax.experimental.pallas import tpu_sc as plsc

_NUM_FIELDS = 26
_VOCAB = 100000
_EMB = 32
_BATCH = 16384
_DENSE = 13
_OUT_W = _NUM_FIELDS * _EMB + _DENSE

_NC = 2
_NS = 16
_NW = _NC * _NS
_B_PER_W = _BATCH // _NW
_CH = 16
_N_CHUNKS = _B_PER_W // _CH


def _body(*refs):
    cats = refs[:_NUM_FIELDS]
    tables = refs[_NUM_FIELDS:2 * _NUM_FIELDS]
    dense = refs[2 * _NUM_FIELDS]
    out = refs[2 * _NUM_FIELDS + 1]
    idx_all = refs[2 * _NUM_FIELDS + 2]
    stage = refs[2 * _NUM_FIELDS + 3]
    dbuf = refs[2 * _NUM_FIELDS + 4]
    dummy = refs[2 * _NUM_FIELDS + 5]
    sem = refs[2 * _NUM_FIELDS + 6]
    wsem = refs[2 * _NUM_FIELDS + 7]

    wid = lax.axis_index("s") * _NC + lax.axis_index("c")

    for f in range(_NUM_FIELDS):
        pltpu.sync_copy(cats[f].at[wid], idx_all.at[f])

    lanes = lax.iota(jnp.int32, 16)
    dmask = lanes < _DENSE

    @pl.loop(0, _N_CHUNKS)
    def _chunk(c):
        base = wid * _B_PER_W + c * _CH
        bank = jnp.bitwise_and(c, 1)

        for f in range(_NUM_FIELDS):
            vec = idx_all[f, pl.ds(c * _CH, 16)]
            for lane in range(16):
                t = vec[lane]
                pltpu.async_copy(
                    tables[f].at[pl.ds(t, 1)],
                    stage.at[bank, pl.ds(lane, 1), pl.ds(f * _EMB, _EMB)],
                    sem)

        pltpu.sync_copy(dense.at[pl.ds(base, _CH)], dbuf)

        @pl.loop(0, _CH)
        def _dk(r):
            r_vec = jnp.full((16,), r, jnp.int32)
            c_vec = jnp.where(dmask, lanes, 0)
            vals = plsc.load_gather(dbuf, [r_vec, c_vec], mask=dmask)
            plsc.store_scatter(stage.at[bank],
                               [r_vec, c_vec + _NUM_FIELDS * _EMB],
                               vals, mask=dmask)

        @pl.when(c >= 1)
        def _():
            pltpu.make_async_copy(
                out.at[pl.ds(0, _CH)], stage.at[1 - bank], wsem).wait()

        pltpu.make_async_copy(
            tables[0].at[pl.ds(0, _CH * _NUM_FIELDS)], dummy, sem).wait()
        pltpu.async_copy(stage.at[bank], out.at[pl.ds(base, _CH)], wsem)

    pltpu.make_async_copy(
        out.at[pl.ds(0, _CH)],
        stage.at[jnp.bitwise_and(_N_CHUNKS - 1, 1)], wsem).wait()


def kernel(cat_0, cat_1, cat_2, cat_3, cat_4, cat_5, cat_6, cat_7, cat_8, cat_9, cat_10, cat_11, cat_12, cat_13, cat_14, cat_15, cat_16, cat_17, cat_18, cat_19, cat_20, cat_21, cat_22, cat_23, cat_24, cat_25, table_0, table_1, table_2, table_3, table_4, table_5, table_6, table_7, table_8, table_9, table_10, table_11, table_12, table_13, table_14, table_15, table_16, table_17, table_18, table_19, table_20, table_21, table_22, table_23, table_24, table_25, dense):
    cats = [cat_0, cat_1, cat_2, cat_3, cat_4, cat_5, cat_6, cat_7, cat_8,
            cat_9, cat_10, cat_11, cat_12, cat_13, cat_14, cat_15, cat_16,
            cat_17, cat_18, cat_19, cat_20, cat_21, cat_22, cat_23, cat_24,
            cat_25]
    tables = [table_0, table_1, table_2, table_3, table_4, table_5, table_6,
              table_7, table_8, table_9, table_10, table_11, table_12,
              table_13, table_14, table_15, table_16, table_17, table_18,
              table_19, table_20, table_21, table_22, table_23, table_24,
              table_25]
    cats_r = [c.reshape(_NW, _B_PER_W) for c in cats]
    dense_r = dense.reshape(_BATCH, _DENSE)

    mesh = plsc.VectorSubcoreMesh(core_axis_name="c", subcore_axis_name="s")
    out = pl.kernel(
        _body,
        out_type=jax.ShapeDtypeStruct((_BATCH, _OUT_W), jnp.float32),
        mesh=mesh,
        scratch_types=[
            pltpu.VMEM((_NUM_FIELDS, _B_PER_W), jnp.int32),
            pltpu.VMEM((2, _CH, _OUT_W), jnp.float32),
            pltpu.VMEM((_CH, _DENSE), jnp.float32),
            pltpu.VMEM((_CH * _NUM_FIELDS, _EMB), jnp.float32),
            pltpu.SemaphoreType.DMA,
            pltpu.SemaphoreType.DMA,
        ],
        compiler_params=pltpu.CompilerParams(
            disable_bounds_checks=True, needs_layout_passes=False,
            use_tc_tiling_on_sc=False),
    )(*cats_r, *tables, dense_r)
    return out.reshape(_BATCH, 1, _OUT_W)
